# baseline (speedup 1.0000x reference)
import functools

import jax
import jax.numpy as jnp
from jax.experimental import pallas as pl

BK = 1024


def _agg_kernel(user_ref, latent_ref, weight_ref, interact_ref, entity_ref,
                out_ref, *, nk):
    k = pl.program_id(0)
    part = jnp.dot(interact_ref[...].astype(jnp.bfloat16),
                   entity_ref[...].astype(jnp.bfloat16),
                   preferred_element_type=jnp.float32)

    @pl.when(k == 0)
    def _init():
        out_ref[...] = part

    @pl.when(k > 0)
    def _acc():
        out_ref[...] += part

    @pl.when(k == nk - 1)
    def _finish():
        score = jnp.dot(user_ref[...], latent_ref[...].T,
                        preferred_element_type=jnp.float32)
        score = jax.nn.softmax(score, axis=1)
        gate = jnp.dot(score, weight_ref[...],
                       preferred_element_type=jnp.float32)
        out_ref[...] *= (1.0 + gate)


@jax.jit
def kernel(entity_emb, user_emb, latent_emb, weight, interact_mat):
    n_users, n_entities = interact_mat.shape
    channel = entity_emb.shape[1]
    nk = n_entities // BK

    return pl.pallas_call(
        functools.partial(_agg_kernel, nk=nk),
        grid=(nk,),
        in_specs=[
            pl.BlockSpec((n_users, channel), lambda k: (0, 0)),
            pl.BlockSpec(latent_emb.shape, lambda k: (0, 0)),
            pl.BlockSpec(weight.shape, lambda k: (0, 0)),
            pl.BlockSpec((n_users, BK), lambda k: (0, k)),
            pl.BlockSpec((BK, channel), lambda k: (k, 0)),
        ],
        out_specs=pl.BlockSpec((n_users, channel), lambda k: (0, 0)),
        out_shape=jax.ShapeDtypeStruct((n_users, channel), jnp.float32),
    )(user_emb, latent_emb, weight, interact_mat, entity_emb)


# speedup vs baseline: 1.0241x; 1.0241x over previous
import functools

import jax
import jax.numpy as jnp
from jax.experimental import pallas as pl
from jax.experimental.pallas import tpu as pltpu

BM = 128
NBUF = 4


def _agg_kernel(user_ref, latent_ref, weight_ref, interact_hbm, entity_ref,
                out_ref, buf_ref, sem, *, nm):
    m = pl.program_id(0)

    @pl.when(m == 0)
    def _warmup():
        for j in range(NBUF):
            pltpu.make_async_copy(
                interact_hbm.at[pl.ds(j * BM, BM), :],
                buf_ref.at[j],
                sem.at[j],
            ).start()

    slot = jax.lax.rem(m, NBUF)
    pltpu.make_async_copy(
        interact_hbm.at[pl.ds(m * BM, BM), :],
        buf_ref.at[slot],
        sem.at[slot],
    ).wait()

    agg = jnp.dot(buf_ref[slot].astype(jnp.bfloat16),
                  entity_ref[...].astype(jnp.bfloat16),
                  preferred_element_type=jnp.float32)
    score = jnp.dot(user_ref[...], latent_ref[...].T,
                    preferred_element_type=jnp.float32)
    score = jax.nn.softmax(score, axis=1)
    gate = jnp.dot(score, weight_ref[...],
                   preferred_element_type=jnp.float32)
    out_ref[...] = agg * (1.0 + gate)

    @pl.when(m + NBUF < nm)
    def _prefetch():
        pltpu.make_async_copy(
            interact_hbm.at[pl.ds((m + NBUF) * BM, BM), :],
            buf_ref.at[slot],
            sem.at[slot],
        ).start()


@jax.jit
def kernel(entity_emb, user_emb, latent_emb, weight, interact_mat):
    n_users, n_entities = interact_mat.shape
    channel = entity_emb.shape[1]
    nm = n_users // BM

    return pl.pallas_call(
        functools.partial(_agg_kernel, nm=nm),
        grid=(nm,),
        in_specs=[
            pl.BlockSpec((BM, channel), lambda m: (m, 0)),
            pl.BlockSpec(latent_emb.shape, lambda m: (0, 0)),
            pl.BlockSpec(weight.shape, lambda m: (0, 0)),
            pl.BlockSpec(memory_space=pltpu.HBM),
            pl.BlockSpec((n_entities, channel), lambda m: (0, 0)),
        ],
        out_specs=pl.BlockSpec((BM, channel), lambda m: (m, 0)),
        out_shape=jax.ShapeDtypeStruct((n_users, channel), jnp.float32),
        scratch_shapes=[
            pltpu.VMEM((NBUF, BM, n_entities), jnp.float32),
            pltpu.SemaphoreType.DMA((NBUF,)),
        ],
    )(user_emb, latent_emb, weight, interact_mat, entity_emb)
